# baseline (speedup 1.0000x reference)
import jax
import jax.numpy as jnp
from jax import lax
from jax.experimental import pallas as pl
from jax.experimental.pallas import tpu as pltpu
from jax.experimental.pallas import tpu_sc as plsc

BATCH = 16384
FIELDS = 100
EMBED_DIM = 64

NC = 2
NS = 16
NW = NC * NS
B_PER_W = BATCH // NW
NBUF = 4
NITER = B_PER_W // NBUF

_WINS = (0, 16, 32, 48, 64, 80, 84)


def _body(x_hbm, w_hbm, out_hbm, idx_raw, idx_dbl, r0, r1, r2, r3,
          g0, g1, g2, g3, o0, o1, o2, o3):
    rows = (r0, r1, r2, r3)
    gsem = (g0, g1, g2, g3)
    osem = (o0, o1, o2, o3)
    wid = lax.axis_index("s") * NC + lax.axis_index("c")
    base = wid * B_PER_W

    pltpu.sync_copy(x_hbm.at[wid], idx_raw)

    def dbl(c, b):
        for st in _WINS:
            v = idx_raw[c, pl.ds(st, 16)]
            idx_dbl[b, pl.ds(st, 16)] = v + v

    def gather(j, b):
        pltpu.async_copy(w_hbm.at[idx_dbl.at[b, pl.ds(0, FIELDS)]], rows[b],
                         gsem[b])

    for b in range(NBUF):
        dbl(b, b)
        gather(b, b)

    def rnd(i, _):
        j0 = i * NBUF
        for b in range(NBUF):
            j = j0 + b
            pltpu.make_async_copy(
                w_hbm.at[idx_dbl.at[b, pl.ds(0, FIELDS)]], rows[b], gsem[b]
            ).wait()
            pltpu.async_copy(rows[b], out_hbm.at[base + j], osem[b])
        for b in range(NBUF):
            j = j0 + b
            pltpu.make_async_copy(rows[b], out_hbm.at[base + j], osem[b]).wait()

            @pl.when(i + 1 < NITER)
            def _():
                dbl(j + NBUF, b)
                gather(j + NBUF, b)

        return 0

    lax.fori_loop(0, NITER, rnd, 0)


@jax.jit
def kernel(x, weight):
    x3 = x.reshape(NW, B_PER_W, FIELDS)
    mesh = plsc.VectorSubcoreMesh(core_axis_name="c", subcore_axis_name="s")
    out = pl.kernel(
        _body,
        mesh=mesh,
        out_type=jax.ShapeDtypeStruct((BATCH, FIELDS, EMBED_DIM), jnp.float32),
        scratch_types=[
            pltpu.VMEM((B_PER_W, FIELDS), jnp.int32),
            pltpu.VMEM((NBUF, FIELDS), jnp.int32),
        ] + [pltpu.VMEM((FIELDS, EMBED_DIM), jnp.float32) for _ in range(NBUF)]
        + [pltpu.SemaphoreType.DMA for _ in range(2 * NBUF)],
        compiler_params=pltpu.CompilerParams(use_tc_tiling_on_sc=False),
    )(x3, weight)
    return out


# speedup vs baseline: 1.0052x; 1.0052x over previous
import jax
import jax.numpy as jnp
from jax import lax
from jax.experimental import pallas as pl
from jax.experimental.pallas import tpu as pltpu
from jax.experimental.pallas import tpu_sc as plsc

BATCH = 16384
FIELDS = 100
EMBED_DIM = 64

NC = 2
NS = 16
NW = NC * NS
B_PER_W = BATCH // NW
NBUF = 8
NITER = B_PER_W // NBUF

_WINS = (0, 16, 32, 48, 64, 80, 84)


def _body(x_hbm, w_hbm, out_hbm, idx_raw, idx_dbl, r0, r1, r2, r3, r4, r5,
          r6, r7, g0, g1, g2, g3, g4, g5, g6, g7, o0, o1, o2, o3, o4, o5,
          o6, o7):
    rows = (r0, r1, r2, r3, r4, r5, r6, r7)
    gsem = (g0, g1, g2, g3, g4, g5, g6, g7)
    osem = (o0, o1, o2, o3, o4, o5, o6, o7)
    wid = lax.axis_index("s") * NC + lax.axis_index("c")
    base = wid * B_PER_W

    pltpu.sync_copy(x_hbm.at[wid], idx_raw)

    def dbl(c, b):
        for st in _WINS:
            v = idx_raw[c, pl.ds(st, 16)]
            idx_dbl[b, pl.ds(st, 16)] = v + v

    def gather(j, b):
        pltpu.async_copy(w_hbm.at[idx_dbl.at[b, pl.ds(0, FIELDS)]], rows[b],
                         gsem[b])

    for b in range(NBUF):
        dbl(b, b)
        gather(b, b)

    def rnd(i, _):
        j0 = i * NBUF
        for b in range(NBUF):
            j = j0 + b
            pltpu.make_async_copy(
                w_hbm.at[idx_dbl.at[b, pl.ds(0, FIELDS)]], rows[b], gsem[b]
            ).wait()
            pltpu.async_copy(rows[b], out_hbm.at[base + j], osem[b])
        for b in range(NBUF):
            j = j0 + b
            pltpu.make_async_copy(rows[b], out_hbm.at[base + j], osem[b]).wait()

            @pl.when(i + 1 < NITER)
            def _():
                dbl(j + NBUF, b)
                gather(j + NBUF, b)

        return 0

    lax.fori_loop(0, NITER, rnd, 0)


@jax.jit
def kernel(x, weight):
    x3 = x.reshape(NW, B_PER_W, FIELDS)
    mesh = plsc.VectorSubcoreMesh(core_axis_name="c", subcore_axis_name="s")
    out = pl.kernel(
        _body,
        mesh=mesh,
        out_type=jax.ShapeDtypeStruct((BATCH, FIELDS, EMBED_DIM), jnp.float32),
        scratch_types=[
            pltpu.VMEM((B_PER_W, FIELDS), jnp.int32),
            pltpu.VMEM((NBUF, FIELDS), jnp.int32),
        ] + [pltpu.VMEM((FIELDS, EMBED_DIM), jnp.float32) for _ in range(NBUF)]
        + [pltpu.SemaphoreType.DMA for _ in range(2 * NBUF)],
        compiler_params=pltpu.CompilerParams(use_tc_tiling_on_sc=False),
    )(x3, weight)
    return out
